# Initial kernel scaffold; baseline (speedup 1.0000x reference)
import jax
import jax.numpy as jnp
from jax import lax
from jax.experimental import pallas as pl
from jax.experimental.pallas import tpu as pltpu
from jax.experimental.pallas import tpu_sc as plsc

N = 10000
NP = 10240
E = 320000
D = 128
G = 64
K = 30
KP = 32
NC, NS = 2, 16
NW = NC * NS
EPW = E // NW
EB = 80
NBLK = EPW // EB
RPW = NP // NW
APT = NP // NS
WMAX = 1024
NSENT = N + 100
NEG = jnp.float32(-jnp.inf)
BIG = jnp.int32(1 << 30)

F32 = jnp.float32
I32 = jnp.int32

_mesh = plsc.VectorSubcoreMesh(core_axis_name="c", subcore_axis_name="s",
                               num_cores=NC, num_subcores=NS)


def _wid():
    return lax.axis_index("s") * NC + lax.axis_index("c")


def _bcast_lane(v16, lane):
    idxs = jnp.full((16, 1), lane, I32)
    dn = lax.GatherDimensionNumbers(offset_dims=(), collapsed_slice_dims=(0,),
                                    start_index_map=(0,))
    return lax.gather(v16, idxs, dn, (1,),
                      mode=lax.GatherScatterMode.PROMISE_IN_BOUNDS)


def _sc_pre_body(z_hbm, emb_hbm, col_hbm, ew_hbm, batch_hbm, z1_hbm,
                 x0_hbm, degp_hbm, cntp_hbm,
                 idx_v, rows_v, ew_v, ones_v, deg_sh, cnt_sh, sem):
    c = lax.axis_index("c")
    s = lax.axis_index("s")
    wid = _wid()
    pltpu.sync_copy(z1_hbm.at[pl.ds(0, APT)], deg_sh.at[pl.ds(s * APT, APT)])

    @pl.when(s == 0)
    def _():
        pltpu.sync_copy(z1_hbm.at[pl.ds(0, 72)], cnt_sh)

    for j in range(EB // 16):
        ones_v[pl.ds(j * 16, 16)] = jnp.full((16,), 1.0, F32)
    plsc.subcore_barrier()

    def emb_blk(b, _):
        base = wid * RPW + b * EB
        pltpu.sync_copy(z_hbm.at[pl.ds(base, EB)], idx_v)
        pltpu.async_copy(emb_hbm.at[idx_v], rows_v, sem).wait()
        pltpu.sync_copy(rows_v, x0_hbm.at[pl.ds(base, EB)])
        return 0

    lax.fori_loop(0, RPW // EB, emb_blk, 0)

    def deg_blk(b, _):
        base = wid * EPW + b * EB
        pltpu.sync_copy(col_hbm.at[pl.ds(base, EB)], idx_v)
        pltpu.sync_copy(ew_hbm.at[pl.ds(base, EB)], ew_v)
        pltpu.sync_copy(ew_v, deg_sh.at[idx_v], add=True)
        return 0

    lax.fori_loop(0, NBLK, deg_blk, 0)

    def cnt_blk(b, _):
        base = wid * RPW + b * EB
        pltpu.sync_copy(batch_hbm.at[pl.ds(base, EB)], idx_v)
        pltpu.sync_copy(ones_v, cnt_sh.at[idx_v], add=True)
        return 0

    lax.fori_loop(0, RPW // EB, cnt_blk, 0)
    plsc.subcore_barrier()

    pltpu.sync_copy(deg_sh.at[pl.ds(s * APT, APT)],
                    degp_hbm.at[c, pl.ds(s * APT, APT)])

    @pl.when(s == 0)
    def _():
        pltpu.sync_copy(cnt_sh, cntp_hbm.at[c])


_sc_pre = pl.kernel(
    _sc_pre_body, mesh=_mesh,
    out_type=[jax.ShapeDtypeStruct((NP, D), F32),
              jax.ShapeDtypeStruct((NC, NP), F32),
              jax.ShapeDtypeStruct((NC, 72), F32)],
    scratch_types=[
        pltpu.VMEM((EB,), I32),
        pltpu.VMEM((EB, D), F32),
        pltpu.VMEM((EB,), F32),
        pltpu.VMEM((EB,), F32),
        pltpu.VMEM_SHARED((NP,), F32),
        pltpu.VMEM_SHARED((72,), F32),
        pltpu.SemaphoreType.DMA,
    ])


def _sc_edge_body(y_hbm, row_hbm, col_hbm, ew_hbm, zrow_hbm,
                  accp_hbm,
                  row_v, col_v, ew_v, rows_v, acc_sh, sem):
    c = lax.axis_index("c")
    s = lax.axis_index("s")
    wid = _wid()
    pltpu.sync_copy(zrow_hbm.at[pl.ds(0, APT)], acc_sh.at[pl.ds(s * APT, APT)])
    plsc.subcore_barrier()

    def blk(b, _):
        base = wid * EPW + b * EB
        pltpu.sync_copy(row_hbm.at[pl.ds(base, EB)], row_v)
        pltpu.sync_copy(col_hbm.at[pl.ds(base, EB)], col_v)
        pltpu.sync_copy(ew_hbm.at[pl.ds(base, EB)], ew_v)
        pltpu.async_copy(y_hbm.at[row_v], rows_v, sem).wait()
        for e in range(EB):
            if e % 16 == 0:
                ew16 = ew_v[pl.ds(e, 16)]
            ce = _bcast_lane(ew16, e % 16)
            for j in range(D // 16):
                rows_v[e, pl.ds(j * 16, 16)] = rows_v[e, pl.ds(j * 16, 16)] * ce
        pltpu.sync_copy(rows_v, acc_sh.at[col_v], add=True)
        return 0

    lax.fori_loop(0, NBLK, blk, 0)
    plsc.subcore_barrier()
    pltpu.sync_copy(acc_sh.at[pl.ds(s * APT, APT)],
                    accp_hbm.at[c, pl.ds(s * APT, APT)])


_sc_edge = pl.kernel(
    _sc_edge_body, mesh=_mesh,
    out_type=jax.ShapeDtypeStruct((NC, NP, D), F32),
    scratch_types=[
        pltpu.VMEM((EB,), I32),
        pltpu.VMEM((EB,), I32),
        pltpu.VMEM((EB,), F32),
        pltpu.VMEM((EB, D), F32),
        pltpu.VMEM_SHARED((NP, D), F32),
        pltpu.SemaphoreType.DMA,
    ])


def _sc_edge1_body(y_hbm, row_hbm, col_hbm, ew_hbm, z1_hbm,
                   accp_hbm,
                   y_v, row_v, col_v, ew_v, val_v, acc_sh, sem):
    c = lax.axis_index("c")
    s = lax.axis_index("s")
    wid = _wid()
    pltpu.sync_copy(z1_hbm.at[pl.ds(0, APT)], acc_sh.at[pl.ds(s * APT, APT)])
    pltpu.sync_copy(y_hbm.at[pl.ds(0, NP)], y_v)
    plsc.subcore_barrier()

    def blk(b, _):
        base = wid * EPW + b * EB
        pltpu.sync_copy(row_hbm.at[pl.ds(base, EB)], row_v)
        pltpu.sync_copy(col_hbm.at[pl.ds(base, EB)], col_v)
        pltpu.sync_copy(ew_hbm.at[pl.ds(base, EB)], ew_v)
        for j in range(EB // 16):
            r16 = row_v[pl.ds(j * 16, 16)]
            g16 = plsc.load_gather(y_v, [r16])
            val_v[pl.ds(j * 16, 16)] = g16 * ew_v[pl.ds(j * 16, 16)]
        pltpu.sync_copy(val_v, acc_sh.at[col_v], add=True)
        return 0

    lax.fori_loop(0, NBLK, blk, 0)
    plsc.subcore_barrier()
    pltpu.sync_copy(acc_sh.at[pl.ds(s * APT, APT)],
                    accp_hbm.at[c, pl.ds(s * APT, APT)])


_sc_edge1 = pl.kernel(
    _sc_edge1_body, mesh=_mesh,
    out_type=jax.ShapeDtypeStruct((NC, NP), F32),
    scratch_types=[
        pltpu.VMEM((NP,), F32),
        pltpu.VMEM((EB,), I32),
        pltpu.VMEM((EB,), I32),
        pltpu.VMEM((EB,), F32),
        pltpu.VMEM((EB,), F32),
        pltpu.VMEM_SHARED((NP,), F32),
        pltpu.SemaphoreType.DMA,
    ])


def _sc_pool_body(x4_hbm, starts_hbm, x1_hbm, x2_hbm, x3_hbm,
                  p1_hbm, p2_hbm, p3_hbm, p4_hbm,
                  starts_v, vals_v, idxb, valb, rows_v, sem):
    wid = _wid()
    pltpu.sync_copy(starts_hbm.at[pl.ds(0, 72)], starts_v)
    iota16 = lax.iota(I32, 16)
    m0 = iota16 == 0

    for gi in range(2):
        g = wid * 2 + gi
        g16 = jnp.full((16,), g, I32)
        start = jnp.max(plsc.load_gather(starts_v, [g16]))
        end = jnp.max(plsc.load_gather(starts_v, [g16 + 1]))
        cnt = end - start
        start8 = (start // 8) * 8
        off = start - start8
        win = off + cnt
        nvec = (win + 15) // 16
        pltpu.sync_copy(x4_hbm.at[pl.ds(start8, WMAX)], vals_v)

        def mask_blk(j, _):
            v = vals_v[pl.ds(j * 16, 16)]
            pid = j * 16 + iota16
            ok = (pid >= off) & (pid < win)
            vals_v[pl.ds(j * 16, 16)] = jnp.where(ok, v, NEG)
            return 0

        lax.fori_loop(0, nvec, mask_blk, 0)

        for j in range(KP // 16):
            idxb[pl.ds(j * 16, 16)] = jnp.full((16,), NSENT, I32)
            valb[pl.ds(j * 16, 16)] = jnp.full((16,), 0.0, F32)

        def sel(k, _):
            def amax(j, m):
                return jnp.maximum(m, vals_v[pl.ds(j * 16, 16)])

            m16 = lax.fori_loop(0, nvec, amax, jnp.full((16,), NEG, F32))
            mv = jnp.max(m16)

            def amin(j, n):
                v = vals_v[pl.ds(j * 16, 16)]
                pid = j * 16 + iota16
                return jnp.minimum(n, jnp.where(v == mv, pid, BIG))

            n16 = lax.fori_loop(0, nvec, amin, jnp.full((16,), BIG, I32))
            imin = jnp.min(n16)
            iw = jnp.minimum(imin, WMAX - 1)
            plsc.store_scatter(vals_v, [jnp.full((16,), iw, I32)],
                               jnp.full((16,), NEG, F32), mask=m0)
            validk = k < cnt
            node = jnp.where(validk, start8 + imin, NSENT)
            val = jnp.where(validk, mv, jnp.float32(0.0))
            plsc.store_scatter(idxb, [jnp.full((16,), k, I32)],
                               jnp.full((16,), node, I32), mask=m0)
            plsc.store_scatter(valb, [jnp.full((16,), k, I32)],
                               jnp.full((16,), val, F32), mask=m0)
            return 0

        lax.fori_loop(0, K, sel, 0)

        pltpu.async_copy(x1_hbm.at[idxb], rows_v, sem).wait()
        pltpu.sync_copy(rows_v, p1_hbm.at[g])
        pltpu.async_copy(x2_hbm.at[idxb], rows_v, sem).wait()
        pltpu.sync_copy(rows_v, p2_hbm.at[g])
        pltpu.async_copy(x3_hbm.at[idxb], rows_v, sem).wait()
        pltpu.sync_copy(rows_v, p3_hbm.at[g])
        pltpu.sync_copy(valb, p4_hbm.at[g])


_sc_pool = pl.kernel(
    _sc_pool_body, mesh=_mesh,
    out_type=[jax.ShapeDtypeStruct((G, KP, D), F32),
              jax.ShapeDtypeStruct((G, KP, D), F32),
              jax.ShapeDtypeStruct((G, KP, D), F32),
              jax.ShapeDtypeStruct((G, KP), F32)],
    scratch_types=[
        pltpu.VMEM((72,), I32),
        pltpu.VMEM((WMAX,), F32),
        pltpu.VMEM((KP,), I32),
        pltpu.VMEM((KP,), F32),
        pltpu.VMEM((KP, D), F32),
        pltpu.SemaphoreType.DMA,
    ])


def _dot(a, b):
    return jnp.dot(a, b, preferred_element_type=F32,
                   precision=lax.Precision.HIGHEST)


def _tc1_body(x0, degp, w0, dis_o, y1_o):
    deg = degp[0] + degp[1] + 1.0
    rows = lax.broadcasted_iota(I32, (NP, 1), 0)
    dis = jnp.where(rows < N, lax.rsqrt(deg), 0.0)
    dis_o[...] = dis
    y1_o[...] = dis * _dot(x0[...], w0[...])


def _tc_mid_body(accp, yprev, dis, b, w, x_o, y_o):
    acc = accp[0] + accp[1]
    rows = lax.broadcasted_iota(I32, (NP, 1), 0)
    x = jnp.tanh(dis[...] * (acc + yprev[...]) + b[...])
    x = jnp.where(rows < N, x, 0.0)
    x_o[...] = x
    y_o[...] = dis[...] * _dot(x, w[...])


def _tc5_body(accp, y4, dis, b3, cntp, x4_o, starts_o):
    acc = accp[0] + accp[1]
    rows = lax.broadcasted_iota(I32, (NP, 1), 0)
    x4 = jnp.tanh(dis[...] * (acc + y4[...]) + b3[...])
    x4_o[...] = jnp.where(rows < N, x4, 0.0)
    cnt = cntp[0] + cntp[1]
    hh = lax.broadcasted_iota(I32, (72, 72), 0)
    gg = lax.broadcasted_iota(I32, (72, 72), 1)
    lt = jnp.where(hh < gg, 1.0, 0.0).astype(F32)
    starts_o[...] = _dot(cnt, lt)


def _tc_head_body(p1, p2, p3, p4, w1a, w1b, w1c, w1d, b1c,
                  w2r, b2c, l1w, l1b, l2w, l2b, out_o):
    r1 = p1[...].reshape(G * KP, D)
    r2 = p2[...].reshape(G * KP, D)
    r3 = p3[...].reshape(G * KP, D)
    r4 = p4[...].reshape(G * KP, 1)
    t = _dot(r1, w1a[...]) + _dot(r2, w1b[...]) + _dot(r3, w1c[...])
    t = t + r4 * w1d[...] + b1c[...]
    t = jnp.maximum(t, 0.0).reshape(G, KP, 16)
    m = [jnp.maximum(t[:, 2 * u, :], t[:, 2 * u + 1, :]) for u in range(15)]
    outs = []
    for tt in range(11):
        u = jnp.concatenate([m[tt + s] for s in range(5)], axis=1)
        outs.append(jnp.maximum(_dot(u, w2r[...]) + b2c[...], 0.0))
    x352 = jnp.concatenate(outs, axis=1)
    h = jnp.maximum(_dot(x352, l1w[...]) + l1b[...], 0.0)
    out_o[...] = _dot(h, l2w[...]) + l2b[...]


def _pc(body, out_shape):
    return pl.pallas_call(body, out_shape=out_shape)


def kernel(z, edge_index, batch, edge_weight, z_emb_table,
           W0, b0, W1, b1, W2, b2, W3, b3,
           conv1_w, conv1_b, conv2_w, conv2_b,
           lin1_w, lin1_b, lin2_w, lin2_b):
    z = z.astype(I32)
    row = edge_index[0].astype(I32)
    col = edge_index[1].astype(I32)
    batch = batch.astype(I32)
    ew = edge_weight.astype(F32)

    zp = jnp.pad(z, (0, NP - N))
    batchp = jnp.pad(batch, (0, NP - N), constant_values=G)
    zeros_rows = jnp.zeros((NP, D), F32)
    zeros_1d = jnp.zeros((NP,), F32)

    x0, degp, cntp = _sc_pre(zp, z_emb_table.astype(F32), col, ew, batchp,
                             zeros_1d)

    dis, y1 = _pc(_tc1_body,
                  [jax.ShapeDtypeStruct((NP, 1), F32),
                   jax.ShapeDtypeStruct((NP, D), F32)])(
        x0, degp.reshape(NC, NP, 1), W0)

    accp1 = _sc_edge(y1, row, col, ew, zeros_rows)
    x1, y2 = _pc(_tc_mid_body,
                 [jax.ShapeDtypeStruct((NP, D), F32),
                  jax.ShapeDtypeStruct((NP, D), F32)])(accp1, y1, dis, b0, W1)

    accp2 = _sc_edge(y2, row, col, ew, zeros_rows)
    x2, y3 = _pc(_tc_mid_body,
                 [jax.ShapeDtypeStruct((NP, D), F32),
                  jax.ShapeDtypeStruct((NP, D), F32)])(accp2, y2, dis, b1, W2)

    accp3 = _sc_edge(y3, row, col, ew, zeros_rows)
    w3p = jnp.pad(W3, ((0, 0), (0, 7)))
    x3, y4p = _pc(_tc_mid_body,
                  [jax.ShapeDtypeStruct((NP, D), F32),
                   jax.ShapeDtypeStruct((NP, 8), F32)])(accp3, y3, dis, b2,
                                                        w3p)

    y4 = y4p[:, 0]
    accp4 = _sc_edge1(y4, row, col, ew, zeros_1d)

    x4c, starts_f = _pc(_tc5_body,
                        [jax.ShapeDtypeStruct((NP, 1), F32),
                         jax.ShapeDtypeStruct((1, 72), F32)])(
        accp4.reshape(NC, NP, 1), y4p[:, 0:1], dis, b3.reshape(1, 1),
        cntp.reshape(NC, 1, 72))

    starts = starts_f.reshape(72).astype(I32)
    x4flat = jnp.pad(x4c[:, 0], (0, WMAX))

    p1, p2, p3, p4 = _sc_pool(x4flat, starts, x1, x2, x3)

    w1 = conv1_w[:, 0, :].T
    w1a, w1b, w1c = w1[0:D], w1[D:2 * D], w1[2 * D:3 * D]
    w1d = w1[3 * D].reshape(1, 16)
    w2r = conv2_w.transpose(2, 1, 0).reshape(80, 32)
    l1wp = lin1_w.reshape(32, 11, 128).transpose(1, 0, 2).reshape(352, 128)

    out = _pc(_tc_head_body, jax.ShapeDtypeStruct((G, 1), F32))(
        p1, p2, p3, p4, w1a, w1b, w1c, w1d, conv1_b.reshape(1, 16),
        w2r, conv2_b.reshape(1, 32), l1wp, lin1_b.reshape(1, 128),
        lin2_w, lin2_b.reshape(1, 1))
    return out


# speedup vs baseline: 1.9214x; 1.9214x over previous
import jax
import jax.numpy as jnp
from jax import lax
from jax.experimental import pallas as pl
from jax.experimental.pallas import tpu as pltpu
from jax.experimental.pallas import tpu_sc as plsc

N = 10000
NP = 10240
E = 320000
D = 128
G = 64
K = 30
KP = 32
NC, NS = 2, 16
NW = NC * NS
EX = E + N
EB = 80
NBLKX = EX // EB
TRX = (NBLKX + NW - 1) // NW
RPW = NP // NW
APT = NP // NS
WMAX = 1024
NSENT = N + 100
NEG = float('-inf')
BIG = 1 << 30

F32 = jnp.float32
I32 = jnp.int32

_sc_cache = {}


def _mesh():
    return plsc.VectorSubcoreMesh(core_axis_name="c", subcore_axis_name="s",
                                  num_cores=NC, num_subcores=NS)


def _wid():
    return lax.axis_index("s") * NC + lax.axis_index("c")


def _bcast_lane(v16, lane):
    idxs = jnp.full((16, 1), lane, I32)
    dn = lax.GatherDimensionNumbers(offset_dims=(), collapsed_slice_dims=(0,),
                                    start_index_map=(0,))
    return lax.gather(v16, idxs, dn, (1,),
                      mode=lax.GatherScatterMode.PROMISE_IN_BOUNDS)


def _sc_pre_body(z_hbm, emb_hbm, batch_hbm, z1_hbm,
                 x0_hbm, cntp_hbm,
                 idx_v, rows_v, ones_v, cnt_sh, sem):
    c = lax.axis_index("c")
    s = lax.axis_index("s")
    wid = _wid()

    @pl.when(s == 0)
    def _():
        pltpu.sync_copy(z1_hbm.at[pl.ds(0, 128)], cnt_sh)

    for j in range(EB // 16):
        ones_v[pl.ds(j * 16, 16)] = jnp.full((16,), 1.0, F32)
    plsc.subcore_barrier()

    def emb_blk(b, _):
        base = wid * RPW + b * EB
        pltpu.sync_copy(z_hbm.at[pl.ds(base, EB)], idx_v)
        pltpu.async_copy(emb_hbm.at[idx_v], rows_v, sem).wait()
        pltpu.sync_copy(rows_v, x0_hbm.at[pl.ds(base, EB)])
        return 0

    lax.fori_loop(0, RPW // EB, emb_blk, 0)

    def cnt_blk(b, _):
        base = wid * RPW + b * EB
        pltpu.sync_copy(batch_hbm.at[pl.ds(base, EB)], idx_v)
        pltpu.sync_copy(ones_v, cnt_sh.at[idx_v], add=True)
        return 0

    lax.fori_loop(0, RPW // EB, cnt_blk, 0)
    plsc.subcore_barrier()

    @pl.when(s == 0)
    def _():
        pltpu.sync_copy(cnt_sh, cntp_hbm.at[c])


def _sc_pre(*args):
    if "pre" not in _sc_cache:
        _sc_cache["pre"] = _mk_sc_pre()
    return _sc_cache["pre"](*args)


def _mk_sc_pre():
  return pl.kernel(
    _sc_pre_body, mesh=_mesh(),
    out_type=[jax.ShapeDtypeStruct((NP, D), F32),
              jax.ShapeDtypeStruct((NC, 128), F32)],
    scratch_types=[
        pltpu.VMEM((EB,), I32),
        pltpu.VMEM((EB, D), F32),
        pltpu.VMEM((EB,), F32),
        pltpu.VMEM_SHARED((128,), F32),
        pltpu.SemaphoreType.DMA,
    ])


def _sc_norm_body(dis_hbm, rows_hbm, cols_hbm, ews_hbm,
                  norm_hbm,
                  r_v, c_v, ew_v, dr_v, dc_v, n_v, sem):
    wid = _wid()

    def blk(t, _):
        bid = t * NW + wid

        @pl.when(bid < NBLKX)
        def _():
            base = bid * EB
            pltpu.sync_copy(rows_hbm.at[pl.ds(base, EB)], r_v)
            pltpu.sync_copy(cols_hbm.at[pl.ds(base, EB)], c_v)
            pltpu.sync_copy(ews_hbm.at[pl.ds(base, EB)], ew_v)
            pltpu.async_copy(dis_hbm.at[r_v], dr_v, sem).wait()
            pltpu.async_copy(dis_hbm.at[c_v], dc_v, sem).wait()
            for j in range(EB // 16):
                sl = pl.ds(j * 16, 16)
                n_v[sl] = (dr_v[sl] * ew_v[sl]) * dc_v[sl]
            pltpu.sync_copy(n_v, norm_hbm.at[pl.ds(base, EB)])
        return 0

    lax.fori_loop(0, TRX, blk, 0)


def _sc_norm(*args):
    if "norm" not in _sc_cache:
        _sc_cache["norm"] = pl.kernel(
            _sc_norm_body, mesh=_mesh(),
            out_type=jax.ShapeDtypeStruct((EX,), F32),
            scratch_types=[
                pltpu.VMEM((EB,), I32),
                pltpu.VMEM((EB,), I32),
                pltpu.VMEM((EB,), F32),
                pltpu.VMEM((EB,), F32),
                pltpu.VMEM((EB,), F32),
                pltpu.VMEM((EB,), F32),
                pltpu.SemaphoreType.DMA,
            ])
    return _sc_cache["norm"](*args)


def _sc_upd_body(xw_hbm, rows_hbm, norm_hbm,
                 upd_hbm,
                 r_v, n_v, rows_v, sem):
    wid = _wid()

    def blk(t, _):
        bid = t * NW + wid

        @pl.when(bid < NBLKX)
        def _():
            base = bid * EB
            pltpu.sync_copy(rows_hbm.at[pl.ds(base, EB)], r_v)
            pltpu.sync_copy(norm_hbm.at[pl.ds(base, EB)], n_v)
            pltpu.async_copy(xw_hbm.at[r_v], rows_v, sem).wait()
            for e in range(EB):
                if e % 16 == 0:
                    n16 = n_v[pl.ds(e, 16)]
                ce = _bcast_lane(n16, e % 16)
                for j in range(D // 16):
                    sl = pl.ds(j * 16, 16)
                    rows_v[e, sl] = rows_v[e, sl] * ce
            pltpu.sync_copy(rows_v, upd_hbm.at[pl.ds(base, EB)])
        return 0

    lax.fori_loop(0, TRX, blk, 0)


def _sc_upd(*args):
    if "upd" not in _sc_cache:
        _sc_cache["upd"] = pl.kernel(
            _sc_upd_body, mesh=_mesh(),
            out_type=jax.ShapeDtypeStruct((EX, D), F32),
            scratch_types=[
                pltpu.VMEM((EB,), I32),
                pltpu.VMEM((EB,), F32),
                pltpu.VMEM((EB, D), F32),
                pltpu.SemaphoreType.DMA,
            ])
    return _sc_cache["upd"](*args)


def _sc_upd1_body(xw_hbm, rows_hbm, norm_hbm,
                  upd_hbm,
                  r_v, n_v, g_v, v_v, sem):
    wid = _wid()

    def blk(t, _):
        bid = t * NW + wid

        @pl.when(bid < NBLKX)
        def _():
            base = bid * EB
            pltpu.sync_copy(rows_hbm.at[pl.ds(base, EB)], r_v)
            pltpu.sync_copy(norm_hbm.at[pl.ds(base, EB)], n_v)
            pltpu.async_copy(xw_hbm.at[r_v], g_v, sem).wait()
            for j in range(EB // 16):
                sl = pl.ds(j * 16, 16)
                v_v[sl] = n_v[sl] * g_v[sl]
            pltpu.sync_copy(v_v, upd_hbm.at[pl.ds(base, EB)])
        return 0

    lax.fori_loop(0, TRX, blk, 0)


def _sc_upd1(*args):
    if "upd1" not in _sc_cache:
        _sc_cache["upd1"] = pl.kernel(
            _sc_upd1_body, mesh=_mesh(),
            out_type=jax.ShapeDtypeStruct((EX,), F32),
            scratch_types=[
                pltpu.VMEM((EB,), I32),
                pltpu.VMEM((EB,), F32),
                pltpu.VMEM((EB,), F32),
                pltpu.VMEM((EB,), F32),
                pltpu.SemaphoreType.DMA,
            ])
    return _sc_cache["upd1"](*args)


def _perm(v16, idx16):
    dn = lax.GatherDimensionNumbers(offset_dims=(), collapsed_slice_dims=(0,),
                                    start_index_map=(0,))
    return lax.gather(v16, idx16[:, None], dn, (1,),
                      mode=lax.GatherScatterMode.PROMISE_IN_BOUNDS)


def _vmax_all(v16, iota16):
    for sh in (8, 4, 2, 1):
        v16 = jnp.maximum(v16, _perm(v16, iota16 ^ sh))
    return v16


def _vmin_all(v16, iota16):
    for sh in (8, 4, 2, 1):
        v16 = jnp.minimum(v16, _perm(v16, iota16 ^ sh))
    return v16


def _sc_pool_body(x4_hbm, starts_hbm, x1_hbm, x2_hbm, x3_hbm,
                  p1_hbm, p2_hbm, p3_hbm, p4_hbm,
                  starts_v, vals_v, idxb, valb, rows_v, sem):
    wid = _wid()
    pltpu.sync_copy(starts_hbm.at[pl.ds(0, 128)], starts_v)
    iota16 = lax.iota(I32, 16)
    m0 = iota16 == 0

    for gi in range(2):
        g = wid * 2 + gi
        c16 = starts_v[pl.ds(g, 16)]
        start = c16[0]
        end = c16[1]
        cnt = end - start
        start8 = (start // 8) * 8
        off = start - start8
        win = off + cnt
        nvec = (win + 15) // 16
        pltpu.sync_copy(x4_hbm.at[pl.ds(start8, WMAX)], vals_v)

        def mask_blk(j, _):
            v = vals_v[pl.ds(j * 16, 16)]
            pid = j * 16 + iota16
            ok = (pid >= off) & (pid < win)
            vals_v[pl.ds(j * 16, 16)] = jnp.where(ok, v, NEG)
            return 0

        lax.fori_loop(0, nvec, mask_blk, 0)

        for j in range(KP // 16):
            idxb[pl.ds(j * 16, 16)] = jnp.full((16,), NSENT, I32)
            valb[pl.ds(j * 16, 16)] = jnp.full((16,), 0.0, F32)

        def sel(k, _):
            def amax(j, m):
                return jnp.maximum(m, vals_v[pl.ds(j * 16, 16)])

            m16 = lax.fori_loop(0, nvec, amax, jnp.full((16,), NEG, F32))
            mv16 = _vmax_all(m16, iota16)

            def amin(j, n):
                v = vals_v[pl.ds(j * 16, 16)]
                pid = j * 16 + iota16
                return jnp.minimum(n, jnp.where(v == mv16, pid, BIG))

            n16 = lax.fori_loop(0, nvec, amin, jnp.full((16,), BIG, I32))
            imin16 = _vmin_all(n16, iota16)
            imin = imin16[0]
            vb = jnp.minimum(imin, WMAX - 16)
            ch = vals_v[pl.ds(vb, 16)]
            vals_v[pl.ds(vb, 16)] = jnp.where(iota16 + vb == imin, NEG, ch)
            validk = k < cnt
            node16 = jnp.where(validk, start8 + imin16, NSENT)
            val16 = jnp.where(validk, mv16, 0.0)
            kb = (k // 16) * 16
            ki = idxb[pl.ds(kb, 16)]
            idxb[pl.ds(kb, 16)] = jnp.where(iota16 + kb == k, node16, ki)
            kv = valb[pl.ds(kb, 16)]
            valb[pl.ds(kb, 16)] = jnp.where(iota16 + kb == k, val16, kv)
            return 0

        lax.fori_loop(0, K, sel, 0)

        pltpu.async_copy(x1_hbm.at[idxb], rows_v, sem).wait()
        pltpu.sync_copy(rows_v, p1_hbm.at[g])
        pltpu.async_copy(x2_hbm.at[idxb], rows_v, sem).wait()
        pltpu.sync_copy(rows_v, p2_hbm.at[g])
        pltpu.async_copy(x3_hbm.at[idxb], rows_v, sem).wait()
        pltpu.sync_copy(rows_v, p3_hbm.at[g])
        pltpu.sync_copy(valb, p4_hbm.at[g])


def _sc_pool(*args):
    if "pool" not in _sc_cache:
        _sc_cache["pool"] = _mk_sc_pool()
    return _sc_cache["pool"](*args)


def _mk_sc_pool():
  return pl.kernel(
    _sc_pool_body, mesh=_mesh(),
    out_type=[jax.ShapeDtypeStruct((G, KP, D), F32),
              jax.ShapeDtypeStruct((G, KP, D), F32),
              jax.ShapeDtypeStruct((G, KP, D), F32),
              jax.ShapeDtypeStruct((G, KP), F32)],
    scratch_types=[
        pltpu.VMEM((128,), I32),
        pltpu.VMEM((WMAX,), F32),
        pltpu.VMEM((KP,), I32),
        pltpu.VMEM((KP,), F32),
        pltpu.VMEM((KP, D), F32),
        pltpu.SemaphoreType.DMA,
    ])


def _dot(a, b):
    return jnp.dot(a, b, preferred_element_type=F32)


def _tc1_body(x0, deg, w0, dis_o, xw_o):
    d = deg[...]
    safe = jnp.where(d > 0, d, 1.0)
    dis_o[...] = jnp.where(d > 0, lax.rsqrt(safe), 0.0)
    xw_o[...] = _dot(x0[...], w0[...])


def _tc_mid_body(acc, b, w, x_o, xw_o):
    x = jnp.tanh(acc[...] + b[...])
    x_o[...] = x
    xw_o[...] = _dot(x, w[...])


def _tc5_body(acc4, b3, cntp, x4_o, starts_o):
    x4_o[...] = jnp.tanh(acc4[...] + b3[...])
    cnt = cntp[0] + cntp[1]
    hh = lax.broadcasted_iota(I32, (128, 128), 0)
    gg = lax.broadcasted_iota(I32, (128, 128), 1)
    lt = jnp.where(hh < gg, 1.0, 0.0).astype(F32)
    starts_o[...] = jnp.dot(cnt, lt, preferred_element_type=F32,
                            precision=lax.Precision.HIGHEST)


def _tc_head_body(p1, p2, p3, p4, w1a, w1b, w1c, w1d, b1c,
                  w2r, b2c, l1w, l1b, l2w, l2b, out_o):
    r1 = p1[...].reshape(G * KP, D)
    r2 = p2[...].reshape(G * KP, D)
    r3 = p3[...].reshape(G * KP, D)
    r4 = p4[...]
    t = _dot(r1, w1a[...]) + _dot(r2, w1b[...]) + _dot(r3, w1c[...])
    t = t + r4 * w1d[...] + b1c[...]
    t = jnp.maximum(t, 0.0).reshape(G, KP, 16)
    m = [jnp.maximum(t[:, 2 * u, :], t[:, 2 * u + 1, :]) for u in range(15)]
    outs = []
    for tt in range(11):
        u = jnp.concatenate([m[tt + s] for s in range(5)], axis=1)
        outs.append(jnp.maximum(_dot(u, w2r[...]) + b2c[...], 0.0))
    x352 = jnp.concatenate(outs, axis=1)
    h = jnp.maximum(_dot(x352, l1w[...]) + l1b[...], 0.0)
    out_o[...] = _dot(h, l2w[...]) + l2b[...]


def _pc(body, out_shape):
    return pl.pallas_call(body, out_shape=out_shape)


_SCAT_DN = lax.ScatterDimensionNumbers(
    update_window_dims=(1,), inserted_window_dims=(0,),
    scatter_dims_to_operand_dims=(0,))


def _sorted_scatter_add(upd, cols_s, width):
    return lax.scatter_add(
        jnp.zeros((N, width), F32), cols_s[:, None], upd, _SCAT_DN,
        indices_are_sorted=True, unique_indices=False,
        mode=lax.GatherScatterMode.PROMISE_IN_BOUNDS)


def kernel(z, edge_index, batch, edge_weight, z_emb_table,
           W0, b0, W1, b1, W2, b2, W3, b3,
           conv1_w, conv1_b, conv2_w, conv2_b,
           lin1_w, lin1_b, lin2_w, lin2_b):
    z = z.astype(I32)
    row = edge_index[0].astype(I32)
    col = edge_index[1].astype(I32)
    batch = batch.astype(I32)
    ew = edge_weight.astype(F32)

    zp = jnp.pad(z, (0, NP - N))
    batchp = jnp.pad(batch, (0, NP - N), constant_values=G)

    x0p, cntp = _sc_pre(zp, z_emb_table.astype(F32), batchp,
                        jnp.zeros((NP,), F32))
    x0 = x0p[:N]

    loop = jnp.arange(N, dtype=I32)
    rowx = jnp.concatenate([row, loop])
    colx = jnp.concatenate([col, loop])
    ewx = jnp.concatenate([ew, jnp.ones((N,), F32)])
    perm = jnp.argsort(colx, stable=True)
    rows_s = rowx[perm]
    cols_s = colx[perm]
    ews_s = ewx[perm]

    deg = jax.ops.segment_sum(ewx, colx, num_segments=N)

    dis, xw = _pc(_tc1_body,
                  [jax.ShapeDtypeStruct((N, 1), F32),
                   jax.ShapeDtypeStruct((N, D), F32)])(
        x0, deg.reshape(N, 1), W0)

    norm_s = _sc_norm(dis.reshape(N), rows_s, cols_s, ews_s)

    xs = []
    for li in range(3):
        upd = _sc_upd(xw, rows_s, norm_s)
        acc = _sorted_scatter_add(upd, cols_s, D)
        w = [W1, W2, jnp.pad(W3, ((0, 0), (0, 7)))][li]
        b = [b0, b1, b2][li]
        od = [D, D, 8][li]
        x, xw = _pc(_tc_mid_body,
                    [jax.ShapeDtypeStruct((N, D), F32),
                     jax.ShapeDtypeStruct((N, od), F32)])(acc, b, w)
        xs.append(x)

    xw4 = xw[:, 0]
    upd4 = _sc_upd1(xw4, rows_s, norm_s)
    acc4 = _sorted_scatter_add(upd4[:, None], cols_s, 1)
    x4c, starts_f = _pc(_tc5_body,
                        [jax.ShapeDtypeStruct((N, 1), F32),
                         jax.ShapeDtypeStruct((1, 128), F32)])(
        acc4, b3.reshape(1, 1), cntp.reshape(NC, 1, 128))

    starts = starts_f.reshape(128).astype(I32)
    x4flat = jnp.pad(x4c[:, 0], (0, NP - N + WMAX))
    x1p = jnp.pad(xs[0], ((0, NP - N), (0, 0)))
    x2p = jnp.pad(xs[1], ((0, NP - N), (0, 0)))
    x3p = jnp.pad(xs[2], ((0, NP - N), (0, 0)))

    p1, p2, p3, p4 = _sc_pool(x4flat, starts, x1p, x2p, x3p)

    w1 = conv1_w[:, 0, :].T
    w1a, w1b, w1c = w1[0:D], w1[D:2 * D], w1[2 * D:3 * D]
    w1d = w1[3 * D].reshape(1, 16)
    w2r = conv2_w.transpose(2, 1, 0).reshape(80, 32)
    l1wp = lin1_w.reshape(32, 11, 128).transpose(1, 0, 2).reshape(352, 128)

    out = _pc(_tc_head_body, jax.ShapeDtypeStruct((G, 1), F32))(
        p1, p2, p3, p4.reshape(G * KP, 1), w1a, w1b, w1c, w1d,
        conv1_b.reshape(1, 16), w2r, conv2_b.reshape(1, 32), l1wp,
        lin1_b.reshape(1, 128), lin2_w, lin2_b.reshape(1, 1))
    return out


# speedup vs baseline: 2.1104x; 1.0984x over previous
import jax
import jax.numpy as jnp
from jax import lax
from jax.experimental import pallas as pl
from jax.experimental.pallas import tpu as pltpu
from jax.experimental.pallas import tpu_sc as plsc

N = 10000
NP = 10240
E = 320000
D = 128
G = 64
K = 30
KP = 32
NC, NS = 2, 16
NW = NC * NS
EX = E + N
EB = 80
NBLKX = EX // EB
TRX = (NBLKX + NW - 1) // NW
RPW = NP // NW
APT = NP // NS
WMAX = 1024
NSENT = N + 100
NEG = float('-inf')
BIG = 1 << 30

F32 = jnp.float32
I32 = jnp.int32

_sc_cache = {}


def _mesh():
    return plsc.VectorSubcoreMesh(core_axis_name="c", subcore_axis_name="s",
                                  num_cores=NC, num_subcores=NS)


def _wid():
    return lax.axis_index("s") * NC + lax.axis_index("c")


def _bcast_lane(v16, lane):
    idxs = jnp.full((16, 1), lane, I32)
    dn = lax.GatherDimensionNumbers(offset_dims=(), collapsed_slice_dims=(0,),
                                    start_index_map=(0,))
    return lax.gather(v16, idxs, dn, (1,),
                      mode=lax.GatherScatterMode.PROMISE_IN_BOUNDS)


def _sc_pre_body(z_hbm, emb_hbm, batch_hbm, z1_hbm,
                 x0_hbm, cntp_hbm,
                 idx_v, rows_v, ones_v, cnt_sh, sem):
    c = lax.axis_index("c")
    s = lax.axis_index("s")
    wid = _wid()

    @pl.when(s == 0)
    def _():
        pltpu.sync_copy(z1_hbm.at[pl.ds(0, 128)], cnt_sh)

    for j in range(EB // 16):
        ones_v[pl.ds(j * 16, 16)] = jnp.full((16,), 1.0, F32)
    plsc.subcore_barrier()

    def emb_blk(b, _):
        base = wid * RPW + b * EB
        pltpu.sync_copy(z_hbm.at[pl.ds(base, EB)], idx_v)
        pltpu.async_copy(emb_hbm.at[idx_v], rows_v, sem).wait()
        pltpu.sync_copy(rows_v, x0_hbm.at[pl.ds(base, EB)])
        return 0

    lax.fori_loop(0, RPW // EB, emb_blk, 0)

    def cnt_blk(b, _):
        base = wid * RPW + b * EB
        pltpu.sync_copy(batch_hbm.at[pl.ds(base, EB)], idx_v)
        pltpu.sync_copy(ones_v, cnt_sh.at[idx_v], add=True)
        return 0

    lax.fori_loop(0, RPW // EB, cnt_blk, 0)
    plsc.subcore_barrier()

    @pl.when(s == 0)
    def _():
        pltpu.sync_copy(cnt_sh, cntp_hbm.at[c])


def _sc_pre(*args):
    if "pre" not in _sc_cache:
        _sc_cache["pre"] = _mk_sc_pre()
    return _sc_cache["pre"](*args)


def _mk_sc_pre():
  return pl.kernel(
    _sc_pre_body, mesh=_mesh(),
    out_type=[jax.ShapeDtypeStruct((NP, D), F32),
              jax.ShapeDtypeStruct((NC, 128), F32)],
    scratch_types=[
        pltpu.VMEM((EB,), I32),
        pltpu.VMEM((EB, D), F32),
        pltpu.VMEM((EB,), F32),
        pltpu.VMEM_SHARED((128,), F32),
        pltpu.SemaphoreType.DMA,
    ])


def _sc_norm_body(dis_hbm, rows_hbm, cols_hbm, ews_hbm,
                  norm_hbm,
                  r_v0, c_v0, ew_v0, dr_v0, dc_v0, n_v0,
                  r_v1, c_v1, ew_v1, dr_v1, dc_v1, n_v1, sem0, sem1):
    wid = _wid()
    rv = (r_v0, r_v1)
    cv = (c_v0, c_v1)
    ev = (ew_v0, ew_v1)
    drv = (dr_v0, dr_v1)
    dcv = (dc_v0, dc_v1)
    nv = (n_v0, n_v1)
    sm = (sem0, sem1)
    NTR = 128

    def issue(p, g):
        base = (p * NW + wid) * EB
        pltpu.sync_copy(rows_hbm.at[pl.ds(base, EB)], rv[g])
        pltpu.sync_copy(cols_hbm.at[pl.ds(base, EB)], cv[g])
        pltpu.sync_copy(ews_hbm.at[pl.ds(base, EB)], ev[g])
        pltpu.make_async_copy(dis_hbm.at[rv[g]], drv[g], sm[g]).start()
        pltpu.make_async_copy(dis_hbm.at[cv[g]], dcv[g], sm[g]).start()

    def compute(g):
        for j in range(EB // 16):
            sl = pl.ds(j * 16, 16)
            nv[g][sl] = (drv[g][sl] * ev[g][sl]) * dcv[g][sl]

    issue(0, 0)

    def outer(o, _):
        for g in range(2):
            p = o * 2 + g

            @pl.when(p < NTR - 1)
            def _():
                issue(p + 1, 1 - g)

            pltpu.make_async_copy(dis_hbm.at[rv[g]], drv[g], sm[g]).wait()
            pltpu.make_async_copy(dis_hbm.at[cv[g]], dcv[g], sm[g]).wait()
            compute(g)
            pltpu.sync_copy(nv[g], norm_hbm.at[pl.ds((p * NW + wid) * EB, EB)])
        return 0

    lax.fori_loop(0, NTR // 2, outer, 0)

    @pl.when(wid < NBLKX - NTR * NW)
    def _():
        base = (NTR * NW + wid) * EB
        pltpu.sync_copy(rows_hbm.at[pl.ds(base, EB)], r_v0)
        pltpu.sync_copy(cols_hbm.at[pl.ds(base, EB)], c_v0)
        pltpu.sync_copy(ews_hbm.at[pl.ds(base, EB)], ew_v0)
        pltpu.async_copy(dis_hbm.at[r_v0], dr_v0, sem0).wait()
        pltpu.async_copy(dis_hbm.at[c_v0], dc_v0, sem0).wait()
        compute(0)
        pltpu.sync_copy(n_v0, norm_hbm.at[pl.ds(base, EB)])


def _sc_norm(*args):
    if "norm" not in _sc_cache:
        _sc_cache["norm"] = pl.kernel(
            _sc_norm_body, mesh=_mesh(),
            out_type=jax.ShapeDtypeStruct((EX,), F32),
            scratch_types=[
                pltpu.VMEM((EB,), I32),
                pltpu.VMEM((EB,), I32),
                pltpu.VMEM((EB,), F32),
                pltpu.VMEM((EB,), F32),
                pltpu.VMEM((EB,), F32),
                pltpu.VMEM((EB,), F32),
                pltpu.VMEM((EB,), I32),
                pltpu.VMEM((EB,), I32),
                pltpu.VMEM((EB,), F32),
                pltpu.VMEM((EB,), F32),
                pltpu.VMEM((EB,), F32),
                pltpu.VMEM((EB,), F32),
                pltpu.SemaphoreType.DMA,
                pltpu.SemaphoreType.DMA,
            ])
    return _sc_cache["norm"](*args)


def _sc_upd_body(xw_hbm, rows_hbm, norm_hbm,
                 upd_hbm,
                 r_v0, n_v0, rows_v0, r_v1, n_v1, rows_v1, sem0, sem1):
    wid = _wid()
    rv = (r_v0, r_v1)
    nv = (n_v0, n_v1)
    bv = (rows_v0, rows_v1)
    sm = (sem0, sem1)
    NTR = 128

    def issue(p, g):
        base = (p * NW + wid) * EB
        pltpu.sync_copy(rows_hbm.at[pl.ds(base, EB)], rv[g])
        pltpu.sync_copy(norm_hbm.at[pl.ds(base, EB)], nv[g])
        pltpu.make_async_copy(xw_hbm.at[rv[g]], bv[g], sm[g]).start()

    def scale(g):
        for e in range(EB):
            if e % 16 == 0:
                n16 = nv[g][pl.ds(e, 16)]
            ce = _bcast_lane(n16, e % 16)
            for j in range(D // 16):
                sl = pl.ds(j * 16, 16)
                bv[g][e, sl] = bv[g][e, sl] * ce

    issue(0, 0)

    def outer(o, _):
        for g in range(2):
            p = o * 2 + g

            @pl.when(p < NTR - 1)
            def _():
                issue(p + 1, 1 - g)

            pltpu.make_async_copy(xw_hbm.at[rv[g]], bv[g], sm[g]).wait()
            scale(g)
            pltpu.sync_copy(bv[g], upd_hbm.at[pl.ds((p * NW + wid) * EB, EB)])
        return 0

    lax.fori_loop(0, NTR // 2, outer, 0)

    @pl.when(wid < NBLKX - NTR * NW)
    def _():
        base = (NTR * NW + wid) * EB
        pltpu.sync_copy(rows_hbm.at[pl.ds(base, EB)], r_v0)
        pltpu.sync_copy(norm_hbm.at[pl.ds(base, EB)], n_v0)
        pltpu.async_copy(xw_hbm.at[r_v0], rows_v0, sem0).wait()
        scale(0)
        pltpu.sync_copy(rows_v0, upd_hbm.at[pl.ds(base, EB)])


def _sc_upd(*args):
    if "upd" not in _sc_cache:
        _sc_cache["upd"] = pl.kernel(
            _sc_upd_body, mesh=_mesh(),
            out_type=jax.ShapeDtypeStruct((EX, D), F32),
            scratch_types=[
                pltpu.VMEM((EB,), I32),
                pltpu.VMEM((EB,), F32),
                pltpu.VMEM((EB, D), F32),
                pltpu.VMEM((EB,), I32),
                pltpu.VMEM((EB,), F32),
                pltpu.VMEM((EB, D), F32),
                pltpu.SemaphoreType.DMA,
                pltpu.SemaphoreType.DMA,
            ])
    return _sc_cache["upd"](*args)


def _sc_upd1_body(xw_hbm, rows_hbm, norm_hbm,
                  upd_hbm,
                  r_v, n_v, g_v, v_v, sem):
    wid = _wid()

    def blk(t, _):
        bid = t * NW + wid

        @pl.when(bid < NBLKX)
        def _():
            base = bid * EB
            pltpu.sync_copy(rows_hbm.at[pl.ds(base, EB)], r_v)
            pltpu.sync_copy(norm_hbm.at[pl.ds(base, EB)], n_v)
            pltpu.async_copy(xw_hbm.at[r_v], g_v, sem).wait()
            for j in range(EB // 16):
                sl = pl.ds(j * 16, 16)
                v_v[sl] = n_v[sl] * g_v[sl]
            pltpu.sync_copy(v_v, upd_hbm.at[pl.ds(base, EB)])
        return 0

    lax.fori_loop(0, TRX, blk, 0)


def _sc_upd1(*args):
    if "upd1" not in _sc_cache:
        _sc_cache["upd1"] = pl.kernel(
            _sc_upd1_body, mesh=_mesh(),
            out_type=jax.ShapeDtypeStruct((EX,), F32),
            scratch_types=[
                pltpu.VMEM((EB,), I32),
                pltpu.VMEM((EB,), F32),
                pltpu.VMEM((EB,), F32),
                pltpu.VMEM((EB,), F32),
                pltpu.SemaphoreType.DMA,
            ])
    return _sc_cache["upd1"](*args)


def _perm(v16, idx16):
    dn = lax.GatherDimensionNumbers(offset_dims=(), collapsed_slice_dims=(0,),
                                    start_index_map=(0,))
    return lax.gather(v16, idx16[:, None], dn, (1,),
                      mode=lax.GatherScatterMode.PROMISE_IN_BOUNDS)


def _vmax_all(v16, iota16):
    for sh in (8, 4, 2, 1):
        v16 = jnp.maximum(v16, _perm(v16, iota16 ^ sh))
    return v16


def _vmin_all(v16, iota16):
    for sh in (8, 4, 2, 1):
        v16 = jnp.minimum(v16, _perm(v16, iota16 ^ sh))
    return v16


def _sc_pool_body(x4_hbm, starts_hbm, x1_hbm, x2_hbm, x3_hbm,
                  p1_hbm, p2_hbm, p3_hbm, p4_hbm,
                  starts_v, vals_v, idxb, valb, rows_v, sem):
    wid = _wid()
    pltpu.sync_copy(starts_hbm.at[pl.ds(0, 128)], starts_v)
    iota16 = lax.iota(I32, 16)
    m0 = iota16 == 0

    for gi in range(2):
        g = wid * 2 + gi
        c16 = starts_v[pl.ds(g, 16)]
        start = c16[0]
        end = c16[1]
        cnt = end - start
        start8 = (start // 8) * 8
        off = start - start8
        win = off + cnt
        nvec = (win + 15) // 16
        pltpu.sync_copy(x4_hbm.at[pl.ds(start8, WMAX)], vals_v)

        def mask_blk(j, _):
            v = vals_v[pl.ds(j * 16, 16)]
            pid = j * 16 + iota16
            ok = (pid >= off) & (pid < win)
            vals_v[pl.ds(j * 16, 16)] = jnp.where(ok, v, NEG)
            return 0

        lax.fori_loop(0, nvec, mask_blk, 0)

        for j in range(KP // 16):
            idxb[pl.ds(j * 16, 16)] = jnp.full((16,), NSENT, I32)
            valb[pl.ds(j * 16, 16)] = jnp.full((16,), 0.0, F32)

        def sel(k, _):
            def amax(j, m):
                return jnp.maximum(m, vals_v[pl.ds(j * 16, 16)])

            m16 = lax.fori_loop(0, nvec, amax, jnp.full((16,), NEG, F32))
            mv16 = _vmax_all(m16, iota16)

            def amin(j, n):
                v = vals_v[pl.ds(j * 16, 16)]
                pid = j * 16 + iota16
                return jnp.minimum(n, jnp.where(v == mv16, pid, BIG))

            n16 = lax.fori_loop(0, nvec, amin, jnp.full((16,), BIG, I32))
            imin16 = _vmin_all(n16, iota16)
            imin = imin16[0]
            vb = jnp.minimum(imin, WMAX - 16)
            ch = vals_v[pl.ds(vb, 16)]
            vals_v[pl.ds(vb, 16)] = jnp.where(iota16 + vb == imin, NEG, ch)
            validk = k < cnt
            node16 = jnp.where(validk, start8 + imin16, NSENT)
            val16 = jnp.where(validk, mv16, 0.0)
            kb = (k // 16) * 16
            ki = idxb[pl.ds(kb, 16)]
            idxb[pl.ds(kb, 16)] = jnp.where(iota16 + kb == k, node16, ki)
            kv = valb[pl.ds(kb, 16)]
            valb[pl.ds(kb, 16)] = jnp.where(iota16 + kb == k, val16, kv)
            return 0

        lax.fori_loop(0, K, sel, 0)

        pltpu.async_copy(x1_hbm.at[idxb], rows_v, sem).wait()
        pltpu.sync_copy(rows_v, p1_hbm.at[g])
        pltpu.async_copy(x2_hbm.at[idxb], rows_v, sem).wait()
        pltpu.sync_copy(rows_v, p2_hbm.at[g])
        pltpu.async_copy(x3_hbm.at[idxb], rows_v, sem).wait()
        pltpu.sync_copy(rows_v, p3_hbm.at[g])
        pltpu.sync_copy(valb, p4_hbm.at[g])


def _sc_pool(*args):
    if "pool" not in _sc_cache:
        _sc_cache["pool"] = _mk_sc_pool()
    return _sc_cache["pool"](*args)


def _mk_sc_pool():
  return pl.kernel(
    _sc_pool_body, mesh=_mesh(),
    out_type=[jax.ShapeDtypeStruct((G, KP, D), F32),
              jax.ShapeDtypeStruct((G, KP, D), F32),
              jax.ShapeDtypeStruct((G, KP, D), F32),
              jax.ShapeDtypeStruct((G, KP), F32)],
    scratch_types=[
        pltpu.VMEM((128,), I32),
        pltpu.VMEM((WMAX,), F32),
        pltpu.VMEM((KP,), I32),
        pltpu.VMEM((KP,), F32),
        pltpu.VMEM((KP, D), F32),
        pltpu.SemaphoreType.DMA,
    ])


def _dot(a, b):
    return jnp.dot(a, b, preferred_element_type=F32)


def _tc1_body(x0, deg, w0, dis_o, xw_o):
    d = deg[...]
    safe = jnp.where(d > 0, d, 1.0)
    dis_o[...] = jnp.where(d > 0, lax.rsqrt(safe), 0.0)
    xw_o[...] = _dot(x0[...], w0[...])


def _tc_mid_body(acc, b, w, x_o, xw_o):
    x = jnp.tanh(acc[...] + b[...])
    x_o[...] = x
    xw_o[...] = _dot(x, w[...])


def _tc5_body(acc4, b3, cntp, x4_o, starts_o):
    x4_o[...] = jnp.tanh(acc4[...] + b3[...])
    cnt = cntp[0] + cntp[1]
    hh = lax.broadcasted_iota(I32, (128, 128), 0)
    gg = lax.broadcasted_iota(I32, (128, 128), 1)
    lt = jnp.where(hh < gg, 1.0, 0.0).astype(F32)
    starts_o[...] = jnp.dot(cnt, lt, preferred_element_type=F32,
                            precision=lax.Precision.HIGHEST)


def _tc_head_body(p1, p2, p3, p4, w1a, w1b, w1c, w1d, b1c,
                  w2r, b2c, l1w, l1b, l2w, l2b, out_o):
    r1 = p1[...].reshape(G * KP, D)
    r2 = p2[...].reshape(G * KP, D)
    r3 = p3[...].reshape(G * KP, D)
    r4 = p4[...]
    t = _dot(r1, w1a[...]) + _dot(r2, w1b[...]) + _dot(r3, w1c[...])
    t = t + r4 * w1d[...] + b1c[...]
    t = jnp.maximum(t, 0.0).reshape(G, KP, 16)
    m = [jnp.maximum(t[:, 2 * u, :], t[:, 2 * u + 1, :]) for u in range(15)]
    outs = []
    for tt in range(11):
        u = jnp.concatenate([m[tt + s] for s in range(5)], axis=1)
        outs.append(jnp.maximum(_dot(u, w2r[...]) + b2c[...], 0.0))
    x352 = jnp.concatenate(outs, axis=1)
    h = jnp.maximum(_dot(x352, l1w[...]) + l1b[...], 0.0)
    out_o[...] = _dot(h, l2w[...]) + l2b[...]


def _pc(body, out_shape):
    return pl.pallas_call(body, out_shape=out_shape)


_SCAT_DN = lax.ScatterDimensionNumbers(
    update_window_dims=(1,), inserted_window_dims=(0,),
    scatter_dims_to_operand_dims=(0,))


def _sorted_scatter_add(upd, cols_s, width):
    return lax.scatter_add(
        jnp.zeros((N, width), F32), cols_s[:, None], upd, _SCAT_DN,
        indices_are_sorted=True, unique_indices=False,
        mode=lax.GatherScatterMode.PROMISE_IN_BOUNDS)


def kernel(z, edge_index, batch, edge_weight, z_emb_table,
           W0, b0, W1, b1, W2, b2, W3, b3,
           conv1_w, conv1_b, conv2_w, conv2_b,
           lin1_w, lin1_b, lin2_w, lin2_b):
    z = z.astype(I32)
    row = edge_index[0].astype(I32)
    col = edge_index[1].astype(I32)
    batch = batch.astype(I32)
    ew = edge_weight.astype(F32)

    zp = jnp.pad(z, (0, NP - N))
    batchp = jnp.pad(batch, (0, NP - N), constant_values=G)

    x0p, cntp = _sc_pre(zp, z_emb_table.astype(F32), batchp,
                        jnp.zeros((NP,), F32))
    x0 = x0p[:N]

    loop = jnp.arange(N, dtype=I32)
    rowx = jnp.concatenate([row, loop])
    colx = jnp.concatenate([col, loop])
    ewx = jnp.concatenate([ew, jnp.ones((N,), F32)])
    perm = jnp.argsort(colx, stable=True)
    rows_s = rowx[perm]
    cols_s = colx[perm]
    ews_s = ewx[perm]

    deg = jax.ops.segment_sum(ewx, colx, num_segments=N)

    dis, xw = _pc(_tc1_body,
                  [jax.ShapeDtypeStruct((N, 1), F32),
                   jax.ShapeDtypeStruct((N, D), F32)])(
        x0, deg.reshape(N, 1), W0)

    norm_s = _sc_norm(dis.reshape(N), rows_s, cols_s, ews_s)

    xs = []
    for li in range(3):
        upd = _sc_upd(xw, rows_s, norm_s)
        acc = _sorted_scatter_add(upd, cols_s, D)
        w = [W1, W2, jnp.pad(W3, ((0, 0), (0, 7)))][li]
        b = [b0, b1, b2][li]
        od = [D, D, 8][li]
        x, xw = _pc(_tc_mid_body,
                    [jax.ShapeDtypeStruct((N, D), F32),
                     jax.ShapeDtypeStruct((N, od), F32)])(acc, b, w)
        xs.append(x)

    xw4 = xw[:, 0]
    upd4 = _sc_upd1(xw4, rows_s, norm_s)
    acc4 = _sorted_scatter_add(upd4[:, None], cols_s, 1)
    x4c, starts_f = _pc(_tc5_body,
                        [jax.ShapeDtypeStruct((N, 1), F32),
                         jax.ShapeDtypeStruct((1, 128), F32)])(
        acc4, b3.reshape(1, 1), cntp.reshape(NC, 1, 128))

    starts = starts_f.reshape(128).astype(I32)
    x4flat = jnp.pad(x4c[:, 0], (0, NP - N + WMAX))
    x1p = jnp.pad(xs[0], ((0, NP - N), (0, 0)))
    x2p = jnp.pad(xs[1], ((0, NP - N), (0, 0)))
    x3p = jnp.pad(xs[2], ((0, NP - N), (0, 0)))

    p1, p2, p3, p4 = _sc_pool(x4flat, starts, x1p, x2p, x3p)

    w1 = conv1_w[:, 0, :].T
    w1a, w1b, w1c = w1[0:D], w1[D:2 * D], w1[2 * D:3 * D]
    w1d = w1[3 * D].reshape(1, 16)
    w2r = conv2_w.transpose(2, 1, 0).reshape(80, 32)
    l1wp = lin1_w.reshape(32, 11, 128).transpose(1, 0, 2).reshape(352, 128)

    out = _pc(_tc_head_body, jax.ShapeDtypeStruct((G, 1), F32))(
        p1, p2, p3, p4.reshape(G * KP, 1), w1a, w1b, w1c, w1d,
        conv1_b.reshape(1, 16), w2r, conv2_b.reshape(1, 32), l1wp,
        lin1_b.reshape(1, 128), lin2_w, lin2_b.reshape(1, 1))
    return out
